# baseline (speedup 1.0000x reference)
import functools

import jax
import jax.numpy as jnp
from jax import lax
from jax.experimental import pallas as pl
from jax.experimental.pallas import tpu as pltpu
from jax.experimental.pallas import tpu_sc as plsc

N_NODES = 10000
D = 128
E = 320000

NC = 2
NS = 16
CH = 80
EPW = E // (NC * NS)
NCHUNK = EPW // CH
ROWS_PT = 624
ZB = 8
REM = N_NODES - NS * ROWS_PT
DEG_W = 128


def _zero_rows(zbuf, acc_sh, s, width_words):
    def fill(i, _):
        for t in range(width_words // 16):
            zbuf[i, pl.ds(t * 16, 16)] = jnp.zeros((16,), jnp.float32)
        return 0
    lax.fori_loop(0, ZB, fill, 0)
    def zdma(z, _):
        pltpu.sync_copy(zbuf, acc_sh.at[pl.ds(s * ROWS_PT + z * ZB, ZB)])
        return 0
    lax.fori_loop(0, ROWS_PT // ZB, zdma, 0)
    @pl.when(s == NS - 1)
    def _():
        for r in range(REM // ZB):
            pltpu.sync_copy(zbuf, acc_sh.at[pl.ds(NS * ROWS_PT + r * ZB, ZB)])


def _write_back(acc_sh, out_hbm, c, s):
    pltpu.sync_copy(acc_sh.at[pl.ds(s * ROWS_PT, ROWS_PT)],
                    out_hbm.at[c, pl.ds(s * ROWS_PT, ROWS_PT)])
    @pl.when(s == NS - 1)
    def _():
        pltpu.sync_copy(acc_sh.at[pl.ds(NS * ROWS_PT, REM)],
                        out_hbm.at[c, pl.ds(NS * ROWS_PT, REM)])


DEG_Q = 8


def _sc_deg_body(dst_hbm, out_hbm, dst_v, ones_v, zbuf, acc_sh, ssem):
    c = lax.axis_index("c")
    s = lax.axis_index("s")
    def fill_ones(i, _):
        for t in range(DEG_W // 16):
            ones_v[i, pl.ds(t * 16, 16)] = jnp.ones((16,), jnp.float32)
        return 0
    lax.fori_loop(0, CH, fill_ones, 0)
    _zero_rows(zbuf, acc_sh, s, DEG_W)
    plsc.subcore_barrier()
    pltpu.sync_copy(dst_hbm.at[c, s], dst_v)
    def fire(j):
        pltpu.async_copy(ones_v, acc_sh.at[dst_v.at[j]], ssem, add=True)
    def drain_one():
        pltpu.make_async_copy(ones_v, acc_sh.at[pl.ds(0, CH)], ssem).wait()
    for j in range(DEG_Q):
        fire(j)
    def chunk(j, _):
        drain_one()
        fire(j + DEG_Q)
        return 0
    lax.fori_loop(0, NCHUNK - DEG_Q, chunk, 0)
    for _ in range(DEG_Q):
        drain_one()
    plsc.subcore_barrier()
    _write_back(acc_sh, out_hbm, c, s)


CHA = 40
NCHA = EPW // CHA
NRB = 6
NIS = 12


def _sc_agg_body(h_hbm, src_hbm, dst_hbm, out_hbm, *scr):
    sbufs = scr[0:NIS]
    dbufs = scr[NIS:2 * NIS]
    rbufs = scr[2 * NIS:2 * NIS + NRB]
    zbuf = scr[2 * NIS + NRB]
    acc_sh = scr[2 * NIS + NRB + 1]
    sems = scr[2 * NIS + NRB + 2:]
    gsems = sems[0:NRB]
    ssems = sems[NRB:2 * NRB]
    isems = sems[2 * NRB:]
    c = lax.axis_index("c")
    s = lax.axis_index("s")
    wbase = ((c * NS) + s) * EPW

    def istart(j, q):
        base = wbase + j * CHA
        pltpu.async_copy(src_hbm.at[pl.ds(base, CHA)], sbufs[q], isems[q])
        pltpu.async_copy(dst_hbm.at[pl.ds(base, CHA)], dbufs[q], isems[q])

    def iwait(q):
        pltpu.make_async_copy(src_hbm.at[pl.ds(0, CHA)], sbufs[q], isems[q]).wait()
        pltpu.make_async_copy(dst_hbm.at[pl.ds(0, CHA)], dbufs[q], isems[q]).wait()

    def gstart(b, q):
        pltpu.async_copy(h_hbm.at[sbufs[q]], rbufs[b], gsems[b])

    def gwait(b):
        pltpu.make_async_copy(h_hbm.at[pl.ds(0, CHA)], rbufs[b], gsems[b]).wait()

    def sstart(b, q):
        pltpu.async_copy(rbufs[b], acc_sh.at[dbufs[q]], ssems[b], add=True)

    def swait(b):
        pltpu.make_async_copy(rbufs[b], acc_sh.at[pl.ds(0, CHA)], ssems[b]).wait()

    _zero_rows(zbuf, acc_sh, s, D)
    plsc.subcore_barrier()

    def pipe_body(J, first):
        for t in range(3):
            gwait(t)
            sstart(t, t)
        for t in range(3):
            if not first:
                swait(t + 3)
            iwait(t + 3)
            gstart(t + 3, t + 3)
            istart(J + 9 + t, t + 9)
        for t in range(3):
            gwait(t + 3)
            sstart(t + 3, t + 3)
        for t in range(3):
            swait(t)
            iwait(t + 6)
            gstart(t, t + 6)
            istart(J + 12 + t, t)
        for t in range(3):
            gwait(t)
            sstart(t, t + 6)
        for t in range(3):
            swait(t + 3)
            iwait(t + 9)
            gstart(t + 3, t + 9)
            istart(J + 15 + t, t + 3)
        for t in range(3):
            gwait(t + 3)
            sstart(t + 3, t + 9)
        for t in range(3):
            swait(t)
            iwait(t)
            gstart(t, t)
            istart(J + 18 + t, t + 6)
        return 0

    for q in range(9):
        istart(q, q)
    for t in range(3):
        iwait(t)
        gstart(t, t)
    pipe_body(0, True)
    lax.fori_loop(1, (NCHA - 10) // 12, lambda m, _: pipe_body(12 * m, False), 0)

    for t in range(3):
        gwait(t)
        sstart(t, t)
    for t in range(3):
        swait(t + 3)
        iwait(t + 3)
        gstart(t + 3, t + 3)
        if t == 0:
            istart(NCHA - 1, 9)
    for t in range(3):
        gwait(t + 3)
        sstart(t + 3, t + 3)
    for t in range(3):
        swait(t)
        iwait(t + 6)
        gstart(t, t + 6)
    for t in range(3):
        gwait(t)
        sstart(t, t + 6)
    for t in range(3):
        swait(t + 3)
    iwait(9)
    gstart(3, 9)
    gwait(3)
    sstart(3, 9)
    for t in range(3):
        swait(t)
    swait(3)

    plsc.subcore_barrier()
    _write_back(acc_sh, out_hbm, c, s)


_sc_deg = pl.kernel(
    _sc_deg_body,
    mesh=plsc.VectorSubcoreMesh(core_axis_name="c", subcore_axis_name="s"),
    out_type=jax.ShapeDtypeStruct((NC, N_NODES, DEG_W), jnp.float32),
    scratch_types=[
        pltpu.VMEM((NCHUNK, CH), jnp.int32),
        pltpu.VMEM((CH, DEG_W), jnp.float32),
        pltpu.VMEM((ZB, DEG_W), jnp.float32),
        pltpu.VMEM_SHARED((N_NODES, DEG_W), jnp.float32),
        pltpu.SemaphoreType.DMA,
    ],
)

_sc_agg = pl.kernel(
    _sc_agg_body,
    mesh=plsc.VectorSubcoreMesh(core_axis_name="c", subcore_axis_name="s"),
    out_type=jax.ShapeDtypeStruct((NC, N_NODES, D), jnp.float32),
    scratch_types=(
        [pltpu.VMEM((CHA,), jnp.int32) for _ in range(NIS)]
        + [pltpu.VMEM((CHA,), jnp.int32) for _ in range(NIS)]
        + [pltpu.VMEM((CHA, D), jnp.float32) for _ in range(NRB)]
        + [pltpu.VMEM((ZB, D), jnp.float32)]
        + [pltpu.VMEM_SHARED((N_NODES, D), jnp.float32)]
        + [pltpu.SemaphoreType.DMA for _ in range(2 * NRB + NIS)]
    ),
)


BM = 1000


def _dense_body(h_ref, a_ref, d_ref, ws_ref, wn_ref, b_ref, o_ref, *, relu):
    agg = a_ref[0] + a_ref[1]
    deg = d_ref[0] + d_ref[1]
    rdeg = 1.0 / jnp.maximum(deg[:, 0:1], 1.0)
    hn = jnp.dot(agg * rdeg, wn_ref[...], preferred_element_type=jnp.float32)
    hs = jnp.dot(h_ref[...], ws_ref[...], preferred_element_type=jnp.float32)
    out = hs + hn + b_ref[...]
    if relu:
        out = jnp.maximum(out, 0.0)
    o_ref[...] = out


def _dense(h, aggp, degp, ws, wn, b, relu):
    return pl.pallas_call(
        functools.partial(_dense_body, relu=relu),
        grid=(N_NODES // BM,),
        in_specs=[
            pl.BlockSpec((BM, D), lambda i: (i, 0)),
            pl.BlockSpec((NC, BM, D), lambda i: (0, i, 0)),
            pl.BlockSpec((NC, BM, DEG_W), lambda i: (0, i, 0)),
            pl.BlockSpec((D, D), lambda i: (0, 0)),
            pl.BlockSpec((D, D), lambda i: (0, 0)),
            pl.BlockSpec((1, D), lambda i: (0, 0)),
        ],
        out_specs=pl.BlockSpec((BM, D), lambda i: (i, 0)),
        out_shape=jax.ShapeDtypeStruct((N_NODES, D), jnp.float32),
    )(h, aggp, degp, ws, wn, b.reshape(1, D))


def kernel(x, edge_index, Ws1, Wn1, b1, Ws2, Wn2, b2, Ws3, Wn3, b3):
    ei = edge_index.astype(jnp.int32)
    src = ei[0]
    dst = ei[1]
    dst4 = dst.reshape(NC, NS, NCHUNK, CH)
    degp = _sc_deg(dst4)
    a = _sc_agg(x, src, dst)
    h = _dense(x, a, degp, Ws1, Wn1, b1, True)
    a = _sc_agg(h, src, dst)
    h = _dense(h, a, degp, Ws2, Wn2, b2, True)
    a = _sc_agg(h, src, dst)
    return _dense(h, a, degp, Ws3, Wn3, b3, False)


# speedup vs baseline: 1.0351x; 1.0351x over previous
import functools

import jax
import jax.numpy as jnp
from jax import lax
from jax.experimental import pallas as pl
from jax.experimental.pallas import tpu as pltpu
from jax.experimental.pallas import tpu_sc as plsc

N_NODES = 10000
D = 128
E = 320000

NC = 2
NS = 16
CH = 80
EPW = E // (NC * NS)
NCHUNK = EPW // CH
ROWS_PT = 624
ZB = 8
REM = N_NODES - NS * ROWS_PT


def _zero_rows(zbuf, acc_sh, s, width_words):
    def fill(i, _):
        for t in range(width_words // 16):
            zbuf[i, pl.ds(t * 16, 16)] = jnp.zeros((16,), jnp.float32)
        return 0
    lax.fori_loop(0, ZB, fill, 0)
    def zdma(z, _):
        pltpu.sync_copy(zbuf, acc_sh.at[pl.ds(s * ROWS_PT + z * ZB, ZB)])
        return 0
    lax.fori_loop(0, ROWS_PT // ZB, zdma, 0)
    @pl.when(s == NS - 1)
    def _():
        for r in range(REM // ZB):
            pltpu.sync_copy(zbuf, acc_sh.at[pl.ds(NS * ROWS_PT + r * ZB, ZB)])


def _write_back(acc_sh, out_hbm, c, s):
    pltpu.sync_copy(acc_sh.at[pl.ds(s * ROWS_PT, ROWS_PT)],
                    out_hbm.at[c, pl.ds(s * ROWS_PT, ROWS_PT)])
    @pl.when(s == NS - 1)
    def _():
        pltpu.sync_copy(acc_sh.at[pl.ds(NS * ROWS_PT, REM)],
                        out_hbm.at[c, pl.ds(NS * ROWS_PT, REM)])


DEG_W = 128
DEG_Q = 8


def _sc_deg_body(dst_hbm, out_hbm, dst_v, ones_v, zbuf, acc_sh, ssem):
    c = lax.axis_index("c")
    s = lax.axis_index("s")
    def fill_ones(i, _):
        for t in range(DEG_W // 16):
            ones_v[i, pl.ds(t * 16, 16)] = jnp.ones((16,), jnp.float32)
        return 0
    lax.fori_loop(0, CH, fill_ones, 0)
    _zero_rows(zbuf, acc_sh, s, DEG_W)
    plsc.subcore_barrier()
    pltpu.sync_copy(dst_hbm.at[c, s], dst_v)
    def fire(j):
        pltpu.async_copy(ones_v, acc_sh.at[dst_v.at[j]], ssem, add=True)
    def drain_one():
        pltpu.make_async_copy(ones_v, acc_sh.at[pl.ds(0, CH)], ssem).wait()
    for j in range(DEG_Q):
        fire(j)
    def chunk(j, _):
        drain_one()
        fire(j + DEG_Q)
        return 0
    lax.fori_loop(0, NCHUNK - DEG_Q, chunk, 0)
    for _ in range(DEG_Q):
        drain_one()
    plsc.subcore_barrier()
    _write_back(acc_sh, out_hbm, c, s)


def _sc_agg_body(h_hbm, src_hbm, dst_hbm, out_hbm,
                 s0, s1, s2, s3, s4, s5, d0, d1, d2, d3, d4, d5,
                 r0, r1, r2, zbuf, acc_sh,
                 gsem0, gsem1, gsem2, ssem0, ssem1, ssem2,
                 isem0, isem1, isem2, isem3, isem4, isem5):
    c = lax.axis_index("c")
    s = lax.axis_index("s")
    sbufs = [s0, s1, s2, s3, s4, s5]
    dbufs = [d0, d1, d2, d3, d4, d5]
    rbufs = [r0, r1, r2]
    gsems = [gsem0, gsem1, gsem2]
    ssems = [ssem0, ssem1, ssem2]
    isems = [isem0, isem1, isem2, isem3, isem4, isem5]
    wbase = ((c * NS) + s) * EPW

    def istart(j, q):
        base = wbase + j * CH
        pltpu.async_copy(src_hbm.at[pl.ds(base, CH)], sbufs[q], isems[q])
        pltpu.async_copy(dst_hbm.at[pl.ds(base, CH)], dbufs[q], isems[q])

    def iwait(q):
        pltpu.make_async_copy(src_hbm.at[pl.ds(0, CH)], sbufs[q], isems[q]).wait()
        pltpu.make_async_copy(dst_hbm.at[pl.ds(0, CH)], dbufs[q], isems[q]).wait()

    def gstart(b, q):
        pltpu.async_copy(h_hbm.at[sbufs[q]], rbufs[b], gsems[b])

    def gwait(b):
        pltpu.make_async_copy(h_hbm.at[pl.ds(0, CH)], rbufs[b], gsems[b]).wait()

    def sstart(b, q):
        pltpu.async_copy(rbufs[b], acc_sh.at[dbufs[q]], ssems[b], add=True)

    def swait(b):
        pltpu.make_async_copy(rbufs[b], acc_sh.at[pl.ds(0, CH)], ssems[b]).wait()

    _zero_rows(zbuf, acc_sh, s, D)
    plsc.subcore_barrier()

    for q in range(6):
        istart(q, q)
    for t in range(3):
        iwait(t)
        gstart(t, t)

    def body(m, _):
        j = 6 * m
        for t in range(3):
            gwait(t)
            sstart(t, t)
        for t in range(3):
            swait(t)
            iwait(t + 3)
            gstart(t, t + 3)
            istart(j + t + 6, t)
        for t in range(3):
            gwait(t)
            sstart(t, t + 3)
        for t in range(3):
            swait(t)
            iwait(t)
            gstart(t, t)
            @pl.when(j + t + 9 < NCHUNK)
            def _():
                istart(j + t + 9, t + 3)
        return 0
    lax.fori_loop(0, (NCHUNK - 5) // 6, body, 0)

    for t in range(3):
        gwait(t)
        sstart(t, t)
    for t in range(2):
        swait(t)
        iwait(t + 3)
        gstart(t, t + 3)
    for t in range(2):
        gwait(t)
        sstart(t, t + 3)
    for t in range(3):
        swait(t)

    plsc.subcore_barrier()
    _write_back(acc_sh, out_hbm, c, s)


_sc_deg = pl.kernel(
    _sc_deg_body,
    mesh=plsc.VectorSubcoreMesh(core_axis_name="c", subcore_axis_name="s"),
    out_type=jax.ShapeDtypeStruct((NC, N_NODES, DEG_W), jnp.float32),
    scratch_types=[
        pltpu.VMEM((NCHUNK, CH), jnp.int32),
        pltpu.VMEM((CH, DEG_W), jnp.float32),
        pltpu.VMEM((ZB, DEG_W), jnp.float32),
        pltpu.VMEM_SHARED((N_NODES, DEG_W), jnp.float32),
        pltpu.SemaphoreType.DMA,
    ],
)

_sc_agg = pl.kernel(
    _sc_agg_body,
    mesh=plsc.VectorSubcoreMesh(core_axis_name="c", subcore_axis_name="s"),
    out_type=jax.ShapeDtypeStruct((NC, N_NODES, D), jnp.float32),
    scratch_types=(
        [pltpu.VMEM((CH,), jnp.int32) for _ in range(6)]
        + [pltpu.VMEM((CH,), jnp.int32) for _ in range(6)]
        + [pltpu.VMEM((CH, D), jnp.float32) for _ in range(3)]
        + [pltpu.VMEM((ZB, D), jnp.float32)]
        + [pltpu.VMEM_SHARED((N_NODES, D), jnp.float32)]
        + [pltpu.SemaphoreType.DMA for _ in range(12)]
    ),
)


BM = 1000
NW = NC * NS


def _dense_body(h_ref, a_ref, d_ref, ws_ref, wn_ref, b_ref, o_ref, *, relu):
    agg = a_ref[0] + a_ref[1]
    deg = d_ref[0] + d_ref[1]
    rdeg = 1.0 / jnp.maximum(deg[:, 0:1], 1.0)
    hn = jnp.dot(agg * rdeg, wn_ref[...], preferred_element_type=jnp.float32)
    hs = jnp.dot(h_ref[...], ws_ref[...], preferred_element_type=jnp.float32)
    out = hs + hn + b_ref[...]
    if relu:
        out = jnp.maximum(out, 0.0)
    o_ref[...] = out


def _dense(h, aggp, degt, ws, wn, b, relu):
    return pl.pallas_call(
        functools.partial(_dense_body, relu=relu),
        grid=(N_NODES // BM,),
        in_specs=[
            pl.BlockSpec((BM, D), lambda i: (i, 0)),
            pl.BlockSpec((NC, BM, D), lambda i: (0, i, 0)),
            pl.BlockSpec((NC, BM, DEG_W), lambda i: (0, i, 0)),
            pl.BlockSpec((D, D), lambda i: (0, 0)),
            pl.BlockSpec((D, D), lambda i: (0, 0)),
            pl.BlockSpec((1, D), lambda i: (0, 0)),
        ],
        out_specs=pl.BlockSpec((BM, D), lambda i: (i, 0)),
        out_shape=jax.ShapeDtypeStruct((N_NODES, D), jnp.float32),
    )(h, aggp, degt, ws, wn, b.reshape(1, D))


def kernel(x, edge_index, Ws1, Wn1, b1, Ws2, Wn2, b2, Ws3, Wn3, b3):
    ei = edge_index.astype(jnp.int32)
    src = ei[0]
    dst = ei[1]
    dst4 = dst.reshape(NC, NS, NCHUNK, CH)
    degp = _sc_deg(dst4)
    a = _sc_agg(x, src, dst)
    h = _dense(x, a, degp, Ws1, Wn1, b1, True)
    a = _sc_agg(h, src, dst)
    h = _dense(h, a, degp, Ws2, Wn2, b2, True)
    a = _sc_agg(h, src, dst)
    return _dense(h, a, degp, Ws3, Wn3, b3, False)


# speedup vs baseline: 1.0410x; 1.0057x over previous
import functools

import jax
import jax.numpy as jnp
from jax import lax
from jax.experimental import pallas as pl
from jax.experimental.pallas import tpu as pltpu
from jax.experimental.pallas import tpu_sc as plsc

N_NODES = 10000
D = 128
E = 320000

NC = 2
NS = 16
CH = 80
EPW = E // (NC * NS)
NCHUNK = EPW // CH
ROWS_PT = 624
ZB = 8
REM = N_NODES - NS * ROWS_PT


def _zero_rows(zbuf, acc_sh, s, width_words):
    def fill(i, _):
        for t in range(width_words // 16):
            zbuf[i, pl.ds(t * 16, 16)] = jnp.zeros((16,), jnp.float32)
        return 0
    lax.fori_loop(0, ZB, fill, 0)
    def zdma(z, _):
        pltpu.sync_copy(zbuf, acc_sh.at[pl.ds(s * ROWS_PT + z * ZB, ZB)])
        return 0
    lax.fori_loop(0, ROWS_PT // ZB, zdma, 0)
    @pl.when(s == NS - 1)
    def _():
        for r in range(REM // ZB):
            pltpu.sync_copy(zbuf, acc_sh.at[pl.ds(NS * ROWS_PT + r * ZB, ZB)])


def _write_back(acc_sh, out_hbm, c, s):
    pltpu.sync_copy(acc_sh.at[pl.ds(s * ROWS_PT, ROWS_PT)],
                    out_hbm.at[c, pl.ds(s * ROWS_PT, ROWS_PT)])
    @pl.when(s == NS - 1)
    def _():
        pltpu.sync_copy(acc_sh.at[pl.ds(NS * ROWS_PT, REM)],
                        out_hbm.at[c, pl.ds(NS * ROWS_PT, REM)])


DEG_W = 128
DEG_Q = 8


def _sc_deg_body(dst_hbm, out_hbm, dst_v, ones_v, zbuf, acc_sh, ssem):
    c = lax.axis_index("c")
    s = lax.axis_index("s")
    def fill_ones(i, _):
        for t in range(DEG_W // 16):
            ones_v[i, pl.ds(t * 16, 16)] = jnp.ones((16,), jnp.float32)
        return 0
    lax.fori_loop(0, CH, fill_ones, 0)
    _zero_rows(zbuf, acc_sh, s, DEG_W)
    plsc.subcore_barrier()
    pltpu.sync_copy(dst_hbm.at[c, s], dst_v)
    def fire(j):
        pltpu.async_copy(ones_v, acc_sh.at[dst_v.at[j]], ssem, add=True)
    def drain_one():
        pltpu.make_async_copy(ones_v, acc_sh.at[pl.ds(0, CH)], ssem).wait()
    for j in range(DEG_Q):
        fire(j)
    def chunk(j, _):
        drain_one()
        fire(j + DEG_Q)
        return 0
    lax.fori_loop(0, NCHUNK - DEG_Q, chunk, 0)
    for _ in range(DEG_Q):
        drain_one()
    plsc.subcore_barrier()
    _write_back(acc_sh, out_hbm, c, s)


def _sc_agg_body(h_hbm, src_hbm, dst_hbm, out_hbm,
                 s0, s1, s2, s3, s4, s5, d0, d1, d2, d3, d4, d5,
                 r0, r1, r2, zbuf, acc_sh,
                 gsem0, gsem1, gsem2, ssem0, ssem1, ssem2,
                 isem0, isem1, isem2, isem3, isem4, isem5):
    c = lax.axis_index("c")
    s = lax.axis_index("s")
    sbufs = [s0, s1, s2, s3, s4, s5]
    dbufs = [d0, d1, d2, d3, d4, d5]
    rbufs = [r0, r1, r2]
    gsems = [gsem0, gsem1, gsem2]
    ssems = [ssem0, ssem1, ssem2]
    isems = [isem0, isem1, isem2, isem3, isem4, isem5]
    wbase = ((c * NS) + s) * EPW

    def istart(j, q):
        base = wbase + j * CH
        pltpu.async_copy(src_hbm.at[pl.ds(base, CH)], sbufs[q], isems[q])
        pltpu.async_copy(dst_hbm.at[pl.ds(base, CH)], dbufs[q], isems[q])

    def iwait(q):
        pltpu.make_async_copy(src_hbm.at[pl.ds(0, CH)], sbufs[q], isems[q]).wait()
        pltpu.make_async_copy(dst_hbm.at[pl.ds(0, CH)], dbufs[q], isems[q]).wait()

    def gstart(b, q):
        pltpu.async_copy(h_hbm.at[sbufs[q]], rbufs[b], gsems[b])

    def gwait(b):
        pltpu.make_async_copy(h_hbm.at[pl.ds(0, CH)], rbufs[b], gsems[b]).wait()

    def sstart(b, q):
        pltpu.async_copy(rbufs[b], acc_sh.at[dbufs[q]], ssems[b], add=True)

    def swait(b):
        pltpu.make_async_copy(rbufs[b], acc_sh.at[pl.ds(0, CH)], ssems[b]).wait()

    _zero_rows(zbuf, acc_sh, s, D)
    plsc.subcore_barrier()

    for q in range(6):
        istart(q, q)
    for t in range(3):
        iwait(t)
        gstart(t, t)

    def body(m, _):
        j = 6 * m
        for t in range(3):
            gwait(t)
            sstart(t, t)
        for t in range(3):
            swait(t)
            iwait(t + 3)
            gstart(t, t + 3)
            istart(j + t + 6, t)
        for t in range(3):
            gwait(t)
            sstart(t, t + 3)
        for t in range(3):
            swait(t)
            iwait(t)
            gstart(t, t)
            @pl.when(j + t + 9 < NCHUNK)
            def _():
                istart(j + t + 9, t + 3)
        return 0
    lax.fori_loop(0, (NCHUNK - 5) // 6, body, 0)

    for t in range(3):
        gwait(t)
        sstart(t, t)
    for t in range(2):
        swait(t)
        iwait(t + 3)
        gstart(t, t + 3)
    for t in range(2):
        gwait(t)
        sstart(t, t + 3)
    for t in range(3):
        swait(t)

    plsc.subcore_barrier()
    _write_back(acc_sh, out_hbm, c, s)


_sc_deg = pl.kernel(
    _sc_deg_body,
    mesh=plsc.VectorSubcoreMesh(core_axis_name="c", subcore_axis_name="s"),
    out_type=jax.ShapeDtypeStruct((NC, N_NODES, DEG_W), jnp.float32),
    scratch_types=[
        pltpu.VMEM((NCHUNK, CH), jnp.int32),
        pltpu.VMEM((CH, DEG_W), jnp.float32),
        pltpu.VMEM((ZB, DEG_W), jnp.float32),
        pltpu.VMEM_SHARED((N_NODES, DEG_W), jnp.float32),
        pltpu.SemaphoreType.DMA,
    ],
)

_sc_agg = pl.kernel(
    _sc_agg_body,
    mesh=plsc.VectorSubcoreMesh(core_axis_name="c", subcore_axis_name="s"),
    out_type=jax.ShapeDtypeStruct((NC, N_NODES, D), jnp.float32),
    scratch_types=(
        [pltpu.VMEM((CH,), jnp.int32) for _ in range(6)]
        + [pltpu.VMEM((CH,), jnp.int32) for _ in range(6)]
        + [pltpu.VMEM((CH, D), jnp.float32) for _ in range(3)]
        + [pltpu.VMEM((ZB, D), jnp.float32)]
        + [pltpu.VMEM_SHARED((N_NODES, D), jnp.float32)]
        + [pltpu.SemaphoreType.DMA for _ in range(12)]
    ),
)


BM = 1000
NW = NC * NS


def _dense_body(h_ref, a_ref, d_ref, ws_ref, wn_ref, b_ref, o_ref, *, relu):
    agg = a_ref[0] + a_ref[1]
    rdeg = d_ref[...]
    hn = jnp.dot(agg * rdeg, wn_ref[...], preferred_element_type=jnp.float32)
    hs = jnp.dot(h_ref[...], ws_ref[...], preferred_element_type=jnp.float32)
    out = hs + hn + b_ref[...]
    if relu:
        out = jnp.maximum(out, 0.0)
    o_ref[...] = out


def _dense(h, aggp, degt, ws, wn, b, relu):
    return pl.pallas_call(
        functools.partial(_dense_body, relu=relu),
        grid=(N_NODES // BM,),
        in_specs=[
            pl.BlockSpec((BM, D), lambda i: (i, 0)),
            pl.BlockSpec((NC, BM, D), lambda i: (0, i, 0)),
            pl.BlockSpec((BM, 1), lambda i: (i, 0)),
            pl.BlockSpec((D, D), lambda i: (0, 0)),
            pl.BlockSpec((D, D), lambda i: (0, 0)),
            pl.BlockSpec((1, D), lambda i: (0, 0)),
        ],
        out_specs=pl.BlockSpec((BM, D), lambda i: (i, 0)),
        out_shape=jax.ShapeDtypeStruct((N_NODES, D), jnp.float32),
    )(h, aggp, degt, ws, wn, b.reshape(1, D))


def kernel(x, edge_index, Ws1, Wn1, b1, Ws2, Wn2, b2, Ws3, Wn3, b3):
    ei = edge_index.astype(jnp.int32)
    src = ei[0]
    dst = ei[1]
    dst4 = dst.reshape(NC, NS, NCHUNK, CH)
    degp = _sc_deg(dst4)
    deg = degp[0, :, 0] + degp[1, :, 0]
    rdeg = (1.0 / jnp.maximum(deg, 1.0))[:, None]
    a = _sc_agg(x, src, dst)
    h = _dense(x, a, rdeg, Ws1, Wn1, b1, True)
    a = _sc_agg(h, src, dst)
    h = _dense(h, a, rdeg, Ws2, Wn2, b2, True)
    a = _sc_agg(h, src, dst)
    return _dense(h, a, rdeg, Ws3, Wn3, b3, False)
